# baseline (speedup 1.0000x reference)
import functools

import jax
import jax.numpy as jnp
from jax import lax
from jax.experimental import pallas as pl
from jax.experimental.pallas import tpu as pltpu
from jax.experimental.pallas import tpu_sc as plsc

U = 5000
I = 5000
N = U + I
D = 128
DH = 64
E = 320000
NC = 2
NS = 16
NW = NC * NS
B = 128
NCH = 80
EP = NW * NCH * B
NCHT = 160
IB = 8
NBLK = NCHT // IB
NP = 10240
RPT = NP // NS
FH = NP // 2
FPT = FH // NS
F32 = jnp.float32


def _spmm_body(curf, colh, rowh, pcol, prow, vals, out,
               b0c, b0r, b0pc, b0pr, b0v, b1c, b1r, b1pc, b1pr, b1v,
               gath0, gath1, spcur, zacc, gsem0, gsem1, isem):
    c = lax.axis_index("c")
    s = lax.axis_index("s")
    ibufs = ((b0c, b0r, b0pc, b0pr, b0v), (b1c, b1r, b1pc, b1pr, b1v))

    def idx_block_copies(ob, bufs):
        base = s * NCHT + ob * IB
        srcs = (colh, rowh, pcol, prow, vals)
        return tuple(
            pltpu.make_async_copy(src.at[pl.ds(base, IB)], dst, isem)
            for src, dst in zip(srcs, bufs))

    def idx_block_start(ob, bufs):
        for d in idx_block_copies(ob, bufs):
            d.start()

    def idx_block_wait(ob, bufs):
        for d in idx_block_copies(ob, bufs):
            d.wait()

    idx_block_start(0, ibufs[0])
    idx_block_start(1, ibufs[1])

    pieces = ((0, B), (B, B), (2 * B, FPT - 2 * B))
    for off, sz in pieces:
        fbase = s * FPT + off
        pltpu.sync_copy(curf.at[pl.ds(c * FH + fbase, sz)],
                        gath0.at[pl.ds(0, sz)])
        pltpu.sync_copy(gath0.at[pl.ds(0, sz)], spcur.at[pl.ds(fbase, sz)])

    @pl.loop(0, B)
    def _zero_g0(r):
        for j in range(D // 16):
            gath0[r, pl.ds(j * 16, 16)] = jnp.zeros((16,), F32)

    for off, sz in pieces:
        pltpu.sync_copy(gath0.at[pl.ds(0, sz)],
                        zacc.at[pl.ds(s * FPT + off, sz)])
    plsc.subcore_barrier()

    def scale(gath, bufs, i):
        _, _, pcb, prb, vb = bufs

        @pl.loop(0, B // 16)
        def _grp(g):
            vvec = vb[i, pl.ds(g * 16, 16)]
            pcvec = pcb[i, pl.ds(g * 16, 16)]
            prvec = prb[i, pl.ds(g * 16, 16)]
            for k in range(16):
                v = vvec[k]
                pc = pcvec[k]
                pr = prvec[k]
                e = g * 16 + k
                for j in range(DH // 16):
                    gath[e, pl.ds(pr + j * 16, 16)] = (
                        gath[e, pl.ds(pc + j * 16, 16)] * v)
                zro = jnp.zeros((16,), F32)
                for j in range(DH // 16):
                    gath[e, pl.ds((DH - pr) + j * 16, 16)] = zro

    def do_block(bufs):
        cb, rb = bufs[0], bufs[1]

        @pl.loop(0, IB // 2)
        def _pair(k):
            i0 = 2 * k
            i1 = i0 + 1
            pltpu.async_copy(spcur.at[cb.at[i1]], gath1, gsem1)
            pltpu.make_async_copy(spcur.at[cb.at[i0]], gath0, gsem0).wait()
            scale(gath0, bufs, i0)
            pltpu.sync_copy(gath0, zacc.at[rb.at[i0]], add=True)

            @pl.when(k < IB // 2 - 1)
            def _():
                pltpu.async_copy(spcur.at[cb.at[i0 + 2]], gath0, gsem0)

            pltpu.make_async_copy(spcur.at[cb.at[i1]], gath1, gsem1).wait()
            scale(gath1, bufs, i1)
            pltpu.sync_copy(gath1, zacc.at[rb.at[i1]], add=True)

    idx_block_wait(0, ibufs[0])
    pltpu.async_copy(spcur.at[b0c.at[0]], gath0, gsem0)

    @pl.loop(0, NBLK // 2)
    def _blkpair(t):
        ob0 = 2 * t
        do_block(ibufs[0])
        idx_block_wait(ob0 + 1, ibufs[1])
        pltpu.async_copy(spcur.at[b1c.at[0]], gath0, gsem0)

        @pl.when(ob0 + 2 < NBLK)
        def _():
            idx_block_start(ob0 + 2, ibufs[0])

        do_block(ibufs[1])

        @pl.when(ob0 + 3 < NBLK)
        def _():
            idx_block_wait(ob0 + 2, ibufs[0])
            pltpu.async_copy(spcur.at[b0c.at[0]], gath0, gsem0)
            idx_block_start(ob0 + 3, ibufs[1])

    plsc.subcore_barrier()
    for off, sz in pieces:
        pltpu.sync_copy(zacc.at[pl.ds(s * FPT + off, sz)],
                        out.at[pl.ds(c * FH + s * FPT + off, sz)])


@functools.cache
def _get_spmm():
    return pl.kernel(
        _spmm_body,
        out_type=jax.ShapeDtypeStruct((NC * FH, D), F32),
        mesh=plsc.VectorSubcoreMesh(core_axis_name="c", subcore_axis_name="s",
                                    num_cores=NC, num_subcores=NS),
        scratch_types=[
            pltpu.VMEM((IB, B), jnp.int32),
            pltpu.VMEM((IB, B), jnp.int32),
            pltpu.VMEM((IB, B), jnp.int32),
            pltpu.VMEM((IB, B), jnp.int32),
            pltpu.VMEM((IB, B), F32),
            pltpu.VMEM((IB, B), jnp.int32),
            pltpu.VMEM((IB, B), jnp.int32),
            pltpu.VMEM((IB, B), jnp.int32),
            pltpu.VMEM((IB, B), jnp.int32),
            pltpu.VMEM((IB, B), F32),
            pltpu.VMEM((B, D), F32),
            pltpu.VMEM((B, D), F32),
            pltpu.VMEM_SHARED((FH, D), F32),
            pltpu.VMEM_SHARED((FH, D), F32),
            pltpu.SemaphoreType.DMA,
            pltpu.SemaphoreType.DMA,
            pltpu.SemaphoreType.DMA,
        ],
    )


def _dense0_body(ego_ref, uw_ref, iw_ref, z_ref,
                 hu_ref, hi_ref, g_ref, ego1_ref):
    ego = ego_ref[...]
    eu = ego[:U]
    ei = ego[U:]
    hu = jnp.dot(eu, uw_ref[...], preferred_element_type=F32)
    hi = jnp.dot(ei, iw_ref[...], preferred_element_type=F32)
    z = z_ref[...]
    lam_u = lax.dot_general(hu, eu, (((0,), (0,)), ((), ())),
                            preferred_element_type=F32)
    lam_i = lax.dot_general(hi, ei, (((0,), (0,)), ((), ())),
                            preferred_element_type=F32)
    g = jnp.concatenate(
        [jnp.dot(hu, lam_u, preferred_element_type=F32),
         jnp.dot(hi, lam_i, preferred_element_type=F32)], axis=0)
    hu_ref[...] = hu
    hi_ref[...] = hi
    g_ref[...] = g
    ego1_ref[...] = (z + g) * 0.5


_dense0 = pl.pallas_call(
    _dense0_body,
    out_shape=(
        jax.ShapeDtypeStruct((U, D), F32),
        jax.ShapeDtypeStruct((I, D), F32),
        jax.ShapeDtypeStruct((N, D), F32),
        jax.ShapeDtypeStruct((N, D), F32),
    ),
)


def _dense1_body(ego0_ref, ego1_ref, hu_ref, hi_ref, z_ref,
                 g_ref, fu_ref, fi_ref):
    ego1 = ego1_ref[...]
    eu = ego1[:U]
    ei = ego1[U:]
    hu = hu_ref[...]
    hi = hi_ref[...]
    z = z_ref[...]
    lam_u = lax.dot_general(hu, eu, (((0,), (0,)), ((), ())),
                            preferred_element_type=F32)
    lam_i = lax.dot_general(hi, ei, (((0,), (0,)), ((), ())),
                            preferred_element_type=F32)
    g = jnp.concatenate(
        [jnp.dot(hu, lam_u, preferred_element_type=F32),
         jnp.dot(hi, lam_i, preferred_element_type=F32)], axis=0)
    ego2 = (z + g) * 0.5
    final = (ego0_ref[...] + ego1 + ego2) * (1.0 / 3.0)
    g_ref[...] = g
    fu_ref[...] = final[:U]
    fi_ref[...] = final[U:]


_dense1 = pl.pallas_call(
    _dense1_body,
    out_shape=(
        jax.ShapeDtypeStruct((N, D), F32),
        jax.ShapeDtypeStruct((U, D), F32),
        jax.ShapeDtypeStruct((I, D), F32),
    ),
)


def _split_flat(x_pad):
    return jnp.concatenate([x_pad[:, :DH].reshape(FH, D),
                            x_pad[:, DH:].reshape(FH, D)])


def _unsplit(zp):
    return jnp.concatenate([zp[:FH].reshape(NP, DH)[:N],
                            zp[FH:].reshape(NP, DH)[:N]], axis=1)


def kernel(user_emb, item_emb, user_hyper_emb, item_hyper_emb,
           adj_indices, adj_values):
    rows = adj_indices[0]
    cols = adj_indices[1]
    ego0 = jnp.concatenate([user_emb, item_emb], axis=0)

    pad = EP - E
    ipad = jnp.zeros((pad,), jnp.int32)
    colp = jnp.concatenate([cols, ipad])
    rowp = jnp.concatenate([rows, ipad])
    sh = (NS * NCHT, B)
    colh = (colp // 2).reshape(sh)
    rowh = (rowp // 2).reshape(sh)
    pcol = ((colp % 2) * DH).reshape(sh)
    prow = ((rowp % 2) * DH).reshape(sh)
    vals2d = jnp.concatenate([adj_values, jnp.zeros((pad,), F32)]
                             ).reshape(sh)
    rpad = jnp.zeros((NP - N, D), F32)

    spmm = _get_spmm()
    zp0 = spmm(_split_flat(jnp.concatenate([ego0, rpad])),
               colh, rowh, pcol, prow, vals2d)
    z0 = _unsplit(zp0)
    hu, hi, g0, ego1 = _dense0(ego0, user_hyper_emb, item_hyper_emb, z0)
    zp1 = spmm(_split_flat(jnp.concatenate([ego1, rpad])),
               colh, rowh, pcol, prow, vals2d)
    z1 = _unsplit(zp1)
    g1, fu, fi = _dense1(ego0, ego1, hu, hi, z1)

    return (fu, fi, (z0, z1), (g0, g1))


# speedup vs baseline: 1.8458x; 1.8458x over previous
import functools

import jax
import jax.numpy as jnp
from jax import lax
from jax.experimental import pallas as pl
from jax.experimental.pallas import tpu as pltpu
from jax.experimental.pallas import tpu_sc as plsc

U = 5000
I = 5000
N = U + I
D = 128
DH = 64
E = 320000
NC = 2
NS = 16
NW = NC * NS
B = 128
NCH = 80
EP = NW * NCH * B
NCHT = 160
IB = 8
NBLK = NCHT // IB
NP = 10240
RPT = NP // NS
FH = NP // 2
FPT = FH // NS
F32 = jnp.float32


def _spmm_body(curf, colh, rowh, fll, flh, fhl, fhh, out,
               b0c, b0r, b0ll, b0lh, b0hl, b0hh,
               b1c, b1r, b1ll, b1lh, b1hl, b1hh,
               gath0, gath1, spcur, zacc, gsem0, gsem1, isem):
    c = lax.axis_index("c")
    s = lax.axis_index("s")
    ibufs = ((b0c, b0r, b0ll, b0lh, b0hl, b0hh),
             (b1c, b1r, b1ll, b1lh, b1hl, b1hh))

    def idx_block_copies(ob, bufs):
        base = s * NCHT + ob * IB
        srcs = (colh, rowh, fll, flh, fhl, fhh)
        return tuple(
            pltpu.make_async_copy(src.at[pl.ds(base, IB)], dst, isem)
            for src, dst in zip(srcs, bufs))

    def idx_block_start(ob, bufs):
        for d in idx_block_copies(ob, bufs):
            d.start()

    def idx_block_wait(ob, bufs):
        for d in idx_block_copies(ob, bufs):
            d.wait()

    idx_block_start(0, ibufs[0])
    idx_block_start(1, ibufs[1])

    pieces = ((0, B), (B, B), (2 * B, FPT - 2 * B))
    for off, sz in pieces:
        fbase = s * FPT + off
        pltpu.sync_copy(curf.at[pl.ds(c * FH + fbase, sz)],
                        gath0.at[pl.ds(0, sz)])
        pltpu.sync_copy(gath0.at[pl.ds(0, sz)], spcur.at[pl.ds(fbase, sz)])

    @pl.loop(0, B)
    def _zero_g0(r):
        for j in range(D // 16):
            gath0[r, pl.ds(j * 16, 16)] = jnp.zeros((16,), F32)

    for off, sz in pieces:
        pltpu.sync_copy(gath0.at[pl.ds(0, sz)],
                        zacc.at[pl.ds(s * FPT + off, sz)])
    plsc.subcore_barrier()

    def scale(gath, bufs, i):
        _, _, bll, blh, bhl, bhh = bufs

        @pl.loop(0, B // 16)
        def _grp(g):
            sl16 = pl.ds(g * 16, 16)
            vll = bll[i, sl16]
            vlh = blh[i, sl16]
            vhl = bhl[i, sl16]
            vhh = bhh[i, sl16]
            for k in range(16):
                cll = vll[k]
                clh = vlh[k]
                chl = vhl[k]
                chh = vhh[k]
                e = g * 16 + k
                for j in range(DH // 16):
                    slo = pl.ds(j * 16, 16)
                    shi = pl.ds(DH + j * 16, 16)
                    lo = gath[e, slo]
                    hi = gath[e, shi]
                    gath[e, slo] = lo * cll + hi * chl
                    gath[e, shi] = lo * clh + hi * chh

    def do_block(bufs):
        cb, rb = bufs[0], bufs[1]

        @pl.loop(0, IB // 2)
        def _pair(k):
            i0 = 2 * k
            i1 = i0 + 1
            pltpu.async_copy(spcur.at[cb.at[i1]], gath1, gsem1)
            pltpu.make_async_copy(spcur.at[cb.at[i0]], gath0, gsem0).wait()
            scale(gath0, bufs, i0)
            pltpu.sync_copy(gath0, zacc.at[rb.at[i0]], add=True)

            @pl.when(k < IB // 2 - 1)
            def _():
                pltpu.async_copy(spcur.at[cb.at[i0 + 2]], gath0, gsem0)

            pltpu.make_async_copy(spcur.at[cb.at[i1]], gath1, gsem1).wait()
            scale(gath1, bufs, i1)
            pltpu.sync_copy(gath1, zacc.at[rb.at[i1]], add=True)

    idx_block_wait(0, ibufs[0])
    pltpu.async_copy(spcur.at[b0c.at[0]], gath0, gsem0)

    @pl.loop(0, NBLK // 2)
    def _blkpair(t):
        ob0 = 2 * t
        do_block(ibufs[0])
        idx_block_wait(ob0 + 1, ibufs[1])
        pltpu.async_copy(spcur.at[b1c.at[0]], gath0, gsem0)

        @pl.when(ob0 + 2 < NBLK)
        def _():
            idx_block_start(ob0 + 2, ibufs[0])

        do_block(ibufs[1])

        @pl.when(ob0 + 3 < NBLK)
        def _():
            idx_block_wait(ob0 + 2, ibufs[0])
            pltpu.async_copy(spcur.at[b0c.at[0]], gath0, gsem0)
            idx_block_start(ob0 + 3, ibufs[1])

    plsc.subcore_barrier()
    for off, sz in pieces:
        pltpu.sync_copy(zacc.at[pl.ds(s * FPT + off, sz)],
                        out.at[pl.ds(c * FH + s * FPT + off, sz)])


@functools.cache
def _get_spmm():
    return pl.kernel(
        _spmm_body,
        out_type=jax.ShapeDtypeStruct((NC * FH, D), F32),
        mesh=plsc.VectorSubcoreMesh(core_axis_name="c", subcore_axis_name="s",
                                    num_cores=NC, num_subcores=NS),
        scratch_types=[
            pltpu.VMEM((IB, B), jnp.int32),
            pltpu.VMEM((IB, B), jnp.int32),
            pltpu.VMEM((IB, B), F32),
            pltpu.VMEM((IB, B), F32),
            pltpu.VMEM((IB, B), F32),
            pltpu.VMEM((IB, B), F32),
            pltpu.VMEM((IB, B), jnp.int32),
            pltpu.VMEM((IB, B), jnp.int32),
            pltpu.VMEM((IB, B), F32),
            pltpu.VMEM((IB, B), F32),
            pltpu.VMEM((IB, B), F32),
            pltpu.VMEM((IB, B), F32),
            pltpu.VMEM((B, D), F32),
            pltpu.VMEM((B, D), F32),
            pltpu.VMEM_SHARED((FH, D), F32),
            pltpu.VMEM_SHARED((FH, D), F32),
            pltpu.SemaphoreType.DMA,
            pltpu.SemaphoreType.DMA,
            pltpu.SemaphoreType.DMA,
        ],
    )


def _dense0_body(ego_ref, uw_ref, iw_ref, z_ref,
                 hu_ref, hi_ref, g_ref, ego1_ref):
    ego = ego_ref[...]
    eu = ego[:U]
    ei = ego[U:]
    hu = jnp.dot(eu, uw_ref[...], preferred_element_type=F32)
    hi = jnp.dot(ei, iw_ref[...], preferred_element_type=F32)
    z = z_ref[...]
    lam_u = lax.dot_general(hu, eu, (((0,), (0,)), ((), ())),
                            preferred_element_type=F32)
    lam_i = lax.dot_general(hi, ei, (((0,), (0,)), ((), ())),
                            preferred_element_type=F32)
    g = jnp.concatenate(
        [jnp.dot(hu, lam_u, preferred_element_type=F32),
         jnp.dot(hi, lam_i, preferred_element_type=F32)], axis=0)
    hu_ref[...] = hu
    hi_ref[...] = hi
    g_ref[...] = g
    ego1_ref[...] = (z + g) * 0.5


_dense0 = pl.pallas_call(
    _dense0_body,
    out_shape=(
        jax.ShapeDtypeStruct((U, D), F32),
        jax.ShapeDtypeStruct((I, D), F32),
        jax.ShapeDtypeStruct((N, D), F32),
        jax.ShapeDtypeStruct((N, D), F32),
    ),
)


def _dense1_body(ego0_ref, ego1_ref, hu_ref, hi_ref, z_ref,
                 g_ref, fu_ref, fi_ref):
    ego1 = ego1_ref[...]
    eu = ego1[:U]
    ei = ego1[U:]
    hu = hu_ref[...]
    hi = hi_ref[...]
    z = z_ref[...]
    lam_u = lax.dot_general(hu, eu, (((0,), (0,)), ((), ())),
                            preferred_element_type=F32)
    lam_i = lax.dot_general(hi, ei, (((0,), (0,)), ((), ())),
                            preferred_element_type=F32)
    g = jnp.concatenate(
        [jnp.dot(hu, lam_u, preferred_element_type=F32),
         jnp.dot(hi, lam_i, preferred_element_type=F32)], axis=0)
    ego2 = (z + g) * 0.5
    final = (ego0_ref[...] + ego1 + ego2) * (1.0 / 3.0)
    g_ref[...] = g
    fu_ref[...] = final[:U]
    fi_ref[...] = final[U:]


_dense1 = pl.pallas_call(
    _dense1_body,
    out_shape=(
        jax.ShapeDtypeStruct((N, D), F32),
        jax.ShapeDtypeStruct((U, D), F32),
        jax.ShapeDtypeStruct((I, D), F32),
    ),
)


def _split_flat(x_pad):
    return jnp.concatenate([x_pad[:, :DH].reshape(FH, D),
                            x_pad[:, DH:].reshape(FH, D)])


def _unsplit(zp):
    return jnp.concatenate([zp[:FH].reshape(NP, DH)[:N],
                            zp[FH:].reshape(NP, DH)[:N]], axis=1)


def kernel(user_emb, item_emb, user_hyper_emb, item_hyper_emb,
           adj_indices, adj_values):
    rows = adj_indices[0]
    cols = adj_indices[1]
    ego0 = jnp.concatenate([user_emb, item_emb], axis=0)

    pad = EP - E
    ipad = jnp.zeros((pad,), jnp.int32)
    colp = jnp.concatenate([cols, ipad])
    rowp = jnp.concatenate([rows, ipad])
    valp = jnp.concatenate([adj_values, jnp.zeros((pad,), F32)])
    sh = (NS * NCHT, B)
    colh = (colp // 2).reshape(sh)
    rowh = (rowp // 2).reshape(sh)
    cp = (colp % 2).astype(F32)
    rp = (rowp % 2).astype(F32)
    fll = (valp * (1 - cp) * (1 - rp)).reshape(sh)
    flh = (valp * (1 - cp) * rp).reshape(sh)
    fhl = (valp * cp * (1 - rp)).reshape(sh)
    fhh = (valp * cp * rp).reshape(sh)
    rpad = jnp.zeros((NP - N, D), F32)

    spmm = _get_spmm()
    zp0 = spmm(_split_flat(jnp.concatenate([ego0, rpad])),
               colh, rowh, fll, flh, fhl, fhh)
    z0 = _unsplit(zp0)
    hu, hi, g0, ego1 = _dense0(ego0, user_hyper_emb, item_hyper_emb, z0)
    zp1 = spmm(_split_flat(jnp.concatenate([ego1, rpad])),
               colh, rowh, fll, flh, fhl, fhh)
    z1 = _unsplit(zp1)
    g1, fu, fi = _dense1(ego0, ego1, hu, hi, z1)

    return (fu, fi, (z0, z1), (g0, g1))
